# baseline (speedup 1.0000x reference)
import functools

import jax
import jax.numpy as jnp
from jax import lax
from jax.experimental import pallas as pl
from jax.experimental.pallas import tpu as pltpu
from jax.experimental.pallas import tpu_sc as plsc

B, L, V, D = 1024, 200, 1000000, 64
N = B * L
DP = 128


@functools.cache
def _make_sc_gather():
    info = plsc.get_sparse_core_info()
    NC, NS = info.num_cores, info.num_subcores
    NW = NC * NS
    per_w = N // NW
    CH = 400
    NCH = per_w // CH

    mesh = plsc.VectorSubcoreMesh(core_axis_name="c", subcore_axis_name="s")

    @functools.partial(
        pl.kernel,
        mesh=mesh,
        out_type=jax.ShapeDtypeStruct((N, DP), jnp.float32),
        scratch_types=[
            pltpu.VMEM((per_w,), jnp.int32),
            pltpu.VMEM((CH, DP), jnp.float32),
            pltpu.VMEM((CH, DP), jnp.float32),
            pltpu.SemaphoreType.DMA,
            pltpu.SemaphoreType.DMA,
            pltpu.SemaphoreType.DMA,
            pltpu.SemaphoreType.DMA,
        ],
        compiler_params=pltpu.CompilerParams(use_tc_tiling_on_sc=False),
    )
    def gather_kernel(idx_hbm, table_hbm, out_hbm, idx_v, rows_a, rows_b,
                      gsem_a, gsem_b, wsem_a, wsem_b):
        wid = lax.axis_index("s") * NC + lax.axis_index("c")
        base = wid * per_w

        pltpu.sync_copy(idx_hbm.at[pl.ds(base, per_w)], idx_v)

        rows = (rows_a, rows_b)
        gsem = (gsem_a, gsem_b)
        wsem = (wsem_a, wsem_b)
        wb = [None, None]
        for c in range(NCH):
            k = c % 2
            if wb[k] is not None:
                wb[k].wait()
            g = pltpu.async_copy(
                table_hbm.at[idx_v.at[pl.ds(c * CH, CH)]], rows[k], gsem[k])
            g.wait()
            wb[k] = pltpu.async_copy(
                rows[k].at[:, pl.ds(0, D)],
                out_hbm.at[pl.ds(base + c * CH, CH), pl.ds(0, D)], wsem[k])
        wb[0].wait()
        wb[1].wait()

    return gather_kernel


_RB = 16384


def _fmt_body(t_ref, o_ref):
    o_ref[:, :D] = jnp.swapaxes(t_ref[...], 0, 1)


def _fmt_call(wt_t):
    nb = (V + _RB - 1) // _RB
    return pl.pallas_call(
        _fmt_body,
        grid=(nb,),
        in_specs=[pl.BlockSpec((D, _RB), lambda i: (0, i))],
        out_specs=pl.BlockSpec((_RB, DP), lambda i: (i, 0)),
        out_shape=jax.ShapeDtypeStruct((V, DP), jnp.float32),
        compiler_params=pltpu.CompilerParams(vmem_limit_bytes=100 * 1024 * 1024),
    )(wt_t)


_LB = 8


def _ln_body(g_ref, sg_ref, pos_ref, par_ref, o_ref):
    x = g_ref[:, :, :D]
    pos = pos_ref[...]
    par = par_ref[...]

    s0 = par[0:1, :].reshape(1, 1, D)
    d1 = (par[1:2, :] - par[0:1, :]).reshape(1, 1, D)
    d2 = (par[2:3, :] - par[1:2, :]).reshape(1, 1, D)
    gam = par[3:4, :].reshape(1, 1, D)
    bet = par[4:5, :].reshape(1, 1, D)

    sg = lax.broadcast_in_dim(sg_ref[...], (_LB, B, D), (0, 1))
    x = x + pos + s0 + jnp.minimum(sg, 1.0) * d1 + jnp.maximum(sg - 1.0, 0.0) * d2

    mu = jnp.mean(x, axis=-1, keepdims=True)
    xc = x - mu
    var = jnp.mean(xc * xc, axis=-1, keepdims=True)
    y = xc * lax.rsqrt(var + 1e-6) * gam + bet
    o_ref[...] = jnp.swapaxes(y, 1, 2)


def _ln_call(gathered, sgf, pos, params):
    return pl.pallas_call(
        _ln_body,
        grid=(L // _LB,),
        in_specs=[
            pl.BlockSpec((_LB, B, DP), lambda i: (i, 0, 0)),
            pl.BlockSpec((_LB, B), lambda i: (i, 0)),
            pl.BlockSpec((_LB, 1, D), lambda i: (i, 0, 0)),
            pl.BlockSpec((8, D), lambda i: (0, 0)),
        ],
        out_specs=pl.BlockSpec((_LB, D, B), lambda i: (i, 0, 0)),
        out_shape=jax.ShapeDtypeStruct((L, D, B), jnp.float32),
        compiler_params=pltpu.CompilerParams(vmem_limit_bytes=100 * 1024 * 1024),
    )(gathered, sgf, pos, params)


def kernel(src, seg, word_table, position_table, segment_table, ln_gamma, ln_beta):
    idx_t = src.astype(jnp.int32).T.reshape(N)
    wt_pad = _fmt_call(word_table.T)
    gathered = _make_sc_gather()(idx_t, wt_pad)
    gathered = gathered.reshape(L, B, DP)

    sgf = seg.astype(jnp.float32).T
    pos = position_table[:L].reshape(L, 1, D)
    params = jnp.concatenate(
        [
            segment_table,
            ln_gamma.reshape(1, D),
            ln_beta.reshape(1, D),
            jnp.zeros((3, D), jnp.float32),
        ],
        axis=0,
    )
    out_t = _ln_call(gathered, sgf, pos, params)
    return out_t.transpose(2, 0, 1)


# speedup vs baseline: 1.0220x; 1.0220x over previous
import functools

import jax
import jax.numpy as jnp
from jax import lax
from jax.experimental import pallas as pl
from jax.experimental.pallas import tpu as pltpu
from jax.experimental.pallas import tpu_sc as plsc

B, L, V, D = 1024, 200, 1000000, 64
N = B * L
DP = 128


@functools.cache
def _make_sc_gather():
    info = plsc.get_sparse_core_info()
    NC, NS = info.num_cores, info.num_subcores
    NW = NC * NS
    per_w = N // NW
    CH = 400
    NCH = per_w // CH

    mesh = plsc.VectorSubcoreMesh(core_axis_name="c", subcore_axis_name="s")

    @functools.partial(
        pl.kernel,
        mesh=mesh,
        out_type=jax.ShapeDtypeStruct((N, DP), jnp.float32),
        scratch_types=[
            pltpu.VMEM((per_w,), jnp.int32),
            pltpu.VMEM((CH, DP), jnp.float32),
            pltpu.VMEM((CH, DP), jnp.float32),
            pltpu.SemaphoreType.DMA,
            pltpu.SemaphoreType.DMA,
            pltpu.SemaphoreType.DMA,
            pltpu.SemaphoreType.DMA,
        ],
        compiler_params=pltpu.CompilerParams(use_tc_tiling_on_sc=False),
    )
    def gather_kernel(idx_hbm, table_hbm, out_hbm, idx_v, rows_a, rows_b,
                      gsem_a, gsem_b, wsem_a, wsem_b):
        wid = lax.axis_index("s") * NC + lax.axis_index("c")
        base = wid * per_w

        pltpu.sync_copy(idx_hbm.at[pl.ds(base, per_w)], idx_v)

        rows = (rows_a, rows_b)
        gsem = (gsem_a, gsem_b)
        wsem = (wsem_a, wsem_b)
        wb = [None, None]
        for c in range(NCH):
            k = c % 2
            if wb[k] is not None:
                wb[k].wait()
            g = pltpu.async_copy(
                table_hbm.at[idx_v.at[pl.ds(c * CH, CH)]], rows[k], gsem[k])
            g.wait()
            wb[k] = pltpu.async_copy(
                rows[k], out_hbm.at[pl.ds(base + c * CH, CH)], wsem[k])
        wb[0].wait()
        wb[1].wait()

    return gather_kernel


_RB = 32768


def _fmt_body(t_ref, o_ref):
    o_ref[:, :D] = jnp.swapaxes(t_ref[...], 0, 1)


def _fmt_call(wt_t):
    nb = (V + _RB - 1) // _RB
    return pl.pallas_call(
        _fmt_body,
        grid=(nb,),
        in_specs=[pl.BlockSpec((D, _RB), lambda i: (0, i))],
        out_specs=pl.BlockSpec((_RB, DP), lambda i: (i, 0)),
        out_shape=jax.ShapeDtypeStruct((V, DP), jnp.float32),
        compiler_params=pltpu.CompilerParams(vmem_limit_bytes=100 * 1024 * 1024),
    )(wt_t)


_LB = 8


def _ln_body(g_ref, sg_ref, pos_ref, par_ref, o_ref):
    x = g_ref[:, :, :D]
    pos = pos_ref[...]
    par = par_ref[...]

    s0 = par[0:1, :].reshape(1, 1, D)
    d1 = (par[1:2, :] - par[0:1, :]).reshape(1, 1, D)
    d2 = (par[2:3, :] - par[1:2, :]).reshape(1, 1, D)
    gam = par[3:4, :].reshape(1, 1, D)
    bet = par[4:5, :].reshape(1, 1, D)

    sg = lax.broadcast_in_dim(sg_ref[...], (_LB, B, D), (0, 1))
    x = x + pos + s0 + jnp.minimum(sg, 1.0) * d1 + jnp.maximum(sg - 1.0, 0.0) * d2

    mu = jnp.mean(x, axis=-1, keepdims=True)
    xc = x - mu
    var = jnp.mean(xc * xc, axis=-1, keepdims=True)
    y = xc * lax.rsqrt(var + 1e-6) * gam + bet
    o_ref[...] = jnp.swapaxes(y, 1, 2)


def _ln_call(gathered, sgf, pos, params):
    return pl.pallas_call(
        _ln_body,
        grid=(L // _LB,),
        in_specs=[
            pl.BlockSpec((_LB, B, DP), lambda i: (i, 0, 0)),
            pl.BlockSpec((_LB, B), lambda i: (i, 0)),
            pl.BlockSpec((_LB, 1, D), lambda i: (i, 0, 0)),
            pl.BlockSpec((8, D), lambda i: (0, 0)),
        ],
        out_specs=pl.BlockSpec((_LB, D, B), lambda i: (i, 0, 0)),
        out_shape=jax.ShapeDtypeStruct((L, D, B), jnp.float32),
        compiler_params=pltpu.CompilerParams(vmem_limit_bytes=100 * 1024 * 1024),
    )(gathered, sgf, pos, params)


def kernel(src, seg, word_table, position_table, segment_table, ln_gamma, ln_beta):
    idx_t = src.astype(jnp.int32).T.reshape(N)
    wt_pad = _fmt_call(word_table.T)
    gathered = _make_sc_gather()(idx_t, wt_pad)
    gathered = gathered.reshape(L, B, DP)

    sgf = seg.astype(jnp.float32).T
    pos = position_table[:L].reshape(L, 1, D)
    params = jnp.concatenate(
        [
            segment_table,
            ln_gamma.reshape(1, D),
            ln_beta.reshape(1, D),
            jnp.zeros((3, D), jnp.float32),
        ],
        axis=0,
    )
    out_t = _ln_call(gathered, sgf, pos, params)
    return out_t.transpose(2, 0, 1)


# speedup vs baseline: 1.0312x; 1.0090x over previous
import functools

import jax
import jax.numpy as jnp
from jax import lax
from jax.experimental import pallas as pl
from jax.experimental.pallas import tpu as pltpu
from jax.experimental.pallas import tpu_sc as plsc

B, L, V, D = 1024, 200, 1000000, 64
N = B * L
DP = 128


@functools.cache
def _make_sc_gather():
    info = plsc.get_sparse_core_info()
    NC, NS = info.num_cores, info.num_subcores
    NW = NC * NS
    per_w = N // NW
    CH = 400
    NCH = per_w // CH

    mesh = plsc.VectorSubcoreMesh(core_axis_name="c", subcore_axis_name="s")

    @functools.partial(
        pl.kernel,
        mesh=mesh,
        out_type=jax.ShapeDtypeStruct((N, DP), jnp.float32),
        scratch_types=[
            pltpu.VMEM((per_w,), jnp.int32),
            pltpu.VMEM((CH, DP), jnp.float32),
            pltpu.VMEM((CH, DP), jnp.float32),
            pltpu.SemaphoreType.DMA,
            pltpu.SemaphoreType.DMA,
            pltpu.SemaphoreType.DMA,
            pltpu.SemaphoreType.DMA,
        ],
        compiler_params=pltpu.CompilerParams(use_tc_tiling_on_sc=False),
    )
    def gather_kernel(idx_hbm, table_hbm, out_hbm, idx_v, rows_a, rows_b,
                      gsem_a, gsem_b, wsem_a, wsem_b):
        wid = lax.axis_index("s") * NC + lax.axis_index("c")
        base = wid * per_w

        pltpu.sync_copy(idx_hbm.at[pl.ds(base, per_w)], idx_v)

        rows = (rows_a, rows_b)
        gsem = (gsem_a, gsem_b)
        wsem = (wsem_a, wsem_b)

        def gather(c, k):
            return pltpu.async_copy(
                table_hbm.at[idx_v.at[pl.ds(c * CH, CH)]], rows[k], gsem[k])

        g = [gather(0, 0), None]
        wb = [None, None]
        for c in range(NCH):
            k = c % 2
            nk = (c + 1) % 2
            if c + 1 < NCH:
                if wb[nk] is not None:
                    wb[nk].wait()
                g[nk] = gather(c + 1, nk)
            g[k].wait()
            wb[k] = pltpu.async_copy(
                rows[k], out_hbm.at[pl.ds(base + c * CH, CH)], wsem[k])
        wb[0].wait()
        wb[1].wait()

    return gather_kernel


_RB = 32768


def _fmt_body(t_ref, o_ref):
    o_ref[:, :D] = jnp.swapaxes(t_ref[...], 0, 1)


def _fmt_call(wt_t):
    nb = (V + _RB - 1) // _RB
    return pl.pallas_call(
        _fmt_body,
        grid=(nb,),
        in_specs=[pl.BlockSpec((D, _RB), lambda i: (0, i))],
        out_specs=pl.BlockSpec((_RB, DP), lambda i: (i, 0)),
        out_shape=jax.ShapeDtypeStruct((V, DP), jnp.float32),
        compiler_params=pltpu.CompilerParams(vmem_limit_bytes=100 * 1024 * 1024),
    )(wt_t)


_LB = 8


def _ln_body(g_ref, sg_ref, pos_ref, par_ref, o_ref):
    x = g_ref[:, :, :D]
    pos = pos_ref[...]
    par = par_ref[...]

    s0 = par[0:1, :].reshape(1, 1, D)
    d1 = (par[1:2, :] - par[0:1, :]).reshape(1, 1, D)
    d2 = (par[2:3, :] - par[1:2, :]).reshape(1, 1, D)
    gam = par[3:4, :].reshape(1, 1, D)
    bet = par[4:5, :].reshape(1, 1, D)

    sg = lax.broadcast_in_dim(sg_ref[...], (_LB, B, D), (0, 1))
    x = x + pos + s0 + jnp.minimum(sg, 1.0) * d1 + jnp.maximum(sg - 1.0, 0.0) * d2

    mu = jnp.mean(x, axis=-1, keepdims=True)
    xc = x - mu
    var = jnp.mean(xc * xc, axis=-1, keepdims=True)
    y = xc * lax.rsqrt(var + 1e-6) * gam + bet
    o_ref[...] = jnp.swapaxes(y, 1, 2)


def _ln_call(gathered, sgf, pos, params):
    return pl.pallas_call(
        _ln_body,
        grid=(L // _LB,),
        in_specs=[
            pl.BlockSpec((_LB, B, DP), lambda i: (i, 0, 0)),
            pl.BlockSpec((_LB, B), lambda i: (i, 0)),
            pl.BlockSpec((_LB, 1, D), lambda i: (i, 0, 0)),
            pl.BlockSpec((8, D), lambda i: (0, 0)),
        ],
        out_specs=pl.BlockSpec((_LB, D, B), lambda i: (i, 0, 0)),
        out_shape=jax.ShapeDtypeStruct((L, D, B), jnp.float32),
        compiler_params=pltpu.CompilerParams(vmem_limit_bytes=100 * 1024 * 1024),
    )(gathered, sgf, pos, params)


def kernel(src, seg, word_table, position_table, segment_table, ln_gamma, ln_beta):
    idx_t = src.astype(jnp.int32).T.reshape(N)
    wt_pad = _fmt_call(word_table.T)
    gathered = _make_sc_gather()(idx_t, wt_pad)
    gathered = gathered.reshape(L, B, DP)

    sgf = seg.astype(jnp.float32).T
    pos = position_table[:L].reshape(L, 1, D)
    params = jnp.concatenate(
        [
            segment_table,
            ln_gamma.reshape(1, D),
            ln_beta.reshape(1, D),
            jnp.zeros((3, D), jnp.float32),
        ],
        axis=0,
    )
    out_t = _ln_call(gathered, sgf, pos, params)
    return out_t.transpose(2, 0, 1)
